# Initial kernel scaffold; baseline (speedup 1.0000x reference)
import jax
import jax.numpy as jnp
from jax.experimental import pallas as pl


def kernel(x, W):
    raise NotImplementedError("write your pallas kernel here")


# speedup vs baseline: 3.4875x; 3.4875x over previous
import functools

import jax
import jax.numpy as jnp
from jax import lax
from jax.experimental import pallas as pl
from jax.experimental.pallas import tpu as pltpu
from jax.experimental.pallas import tpu_sc as plsc

_NE = 8192
_D = 32
_N = 8192
_BETA = 0.25

_R = 1024
_C = 1024
_NR = _N // _R
_NJ = _NE // _C

_NC = 2
_NS = 16
_NW = _NC * _NS
_BPW = _N // _NW
_KCH = 128
_NCH = _BPW // _KCH


def _argmin_body(w_ref, xt_ref, idx_ref, minval, minidx):
    j = pl.program_id(1)

    @pl.when(j == 0)
    def _init():
        minval[...] = jnp.full((1, _R), jnp.inf, jnp.float32)
        minidx[...] = jnp.zeros((1, _R), jnp.int32)

    w = w_ref[...]
    xt = xt_ref[...]
    sim = jnp.dot(w, xt, preferred_element_type=jnp.float32)
    s2 = jnp.sum(w * w, axis=1, keepdims=True)
    dist = s2 - 2.0 * sim
    bmin = jnp.min(dist, axis=0, keepdims=True)
    ids = lax.broadcasted_iota(jnp.int32, (_C, _R), 0) + j * _C
    bidx = jnp.min(jnp.where(dist == bmin, ids, jnp.int32(2 ** 30)),
                   axis=0, keepdims=True)
    prev = minval[...]
    better = bmin < prev
    minidx[...] = jnp.where(better, bidx, minidx[...])
    minval[...] = jnp.where(better, bmin, prev)

    @pl.when(j == _NJ - 1)
    def _fin():
        idx_ref[...] = minidx[...].reshape(1, 1, _R)


_argmin_call = pl.pallas_call(
    _argmin_body,
    grid=(_NR, _NJ),
    in_specs=[
        pl.BlockSpec((_C, _D), lambda r, j: (j, 0)),
        pl.BlockSpec((_D, _R), lambda r, j: (0, r)),
    ],
    out_specs=pl.BlockSpec((1, 1, _R), lambda r, j: (r, 0, 0)),
    out_shape=jax.ShapeDtypeStruct((_NR, 1, _R), jnp.int32),
    scratch_shapes=[
        pltpu.VMEM((1, _R), jnp.float32),
        pltpu.VMEM((1, _R), jnp.int32),
    ],
)


def _gather_body(w_hbm, idx_hbm, x_hbm, q_out, o_out, part_out,
                 idx_v, q_v, x_v, o_v, p_v, sem):
    wid = lax.axis_index("s") * _NC + lax.axis_index("c")
    base = wid * _BPW

    pltpu.sync_copy(idx_hbm.at[pl.ds(wid * _NCH, _NCH), :], idx_v)
    copies = [
        pltpu.async_copy(w_hbm.at[idx_v.at[k]],
                         q_v.at[pl.ds(k * _KCH, _KCH), :], sem)
        for k in range(_NCH)
    ]
    pltpu.sync_copy(x_hbm.at[pl.ds(base, _BPW), :], x_v)
    for cp in copies:
        cp.wait()

    def body(r, acc):
        for c in range(_D // 16):
            sl = pl.ds(c * 16, 16)
            q16 = q_v[r, sl]
            x16 = x_v[r, sl]
            e = q16 - x16
            o_v[r, sl] = x16 + e
            acc = acc + e * e
        return acc

    acc = lax.fori_loop(0, _BPW, body, jnp.zeros((16,), jnp.float32))
    p_v[...] = acc
    pltpu.sync_copy(q_v, q_out.at[pl.ds(base, _BPW), :])
    pltpu.sync_copy(o_v, o_out.at[pl.ds(base, _BPW), :])
    pltpu.sync_copy(p_v, part_out.at[wid])


@functools.cache
def _gather_call():
    return functools.partial(
        pl.kernel,
        out_type=(
            jax.ShapeDtypeStruct((_N, _D), jnp.float32),
            jax.ShapeDtypeStruct((_N, _D), jnp.float32),
            jax.ShapeDtypeStruct((_NW, 16), jnp.float32),
        ),
        mesh=plsc.VectorSubcoreMesh(core_axis_name="c", subcore_axis_name="s",
                                    num_cores=_NC, num_subcores=_NS),
        scratch_types=[
            pltpu.VMEM((_NCH, _KCH), jnp.int32),
            pltpu.VMEM((_BPW, _D), jnp.float32),
            pltpu.VMEM((_BPW, _D), jnp.float32),
            pltpu.VMEM((_BPW, _D), jnp.float32),
            pltpu.VMEM((16,), jnp.float32),
            pltpu.SemaphoreType.DMA,
        ],
        compiler_params=pltpu.CompilerParams(use_tc_tiling_on_sc=False),
    )(_gather_body)


def kernel(x, W):
    x_flat = x.reshape(_N, _D)
    xt = x_flat.T
    idx = _argmin_call(W, xt).reshape(_N)
    idx2 = idx.reshape(_NW * _NCH, _KCH)
    q_flat, o_flat, parts = _gather_call()(W, idx2, x_flat)
    loss = (1.0 + _BETA) * jnp.sum(parts) / (_N * _D)
    quantized = q_flat.reshape(x.shape)
    out = o_flat.reshape(x.shape)
    return (out, quantized, loss)


# speedup vs baseline: 4.0089x; 1.1495x over previous
import functools

import jax
import jax.numpy as jnp
from jax import lax
from jax.experimental import pallas as pl
from jax.experimental.pallas import tpu as pltpu
from jax.experimental.pallas import tpu_sc as plsc

_NE = 8192
_D = 32
_N = 8192
_BETA = 0.25

_R = 2048
_C = 1024
_NR = _N // _R
_NJ = _NE // _C

_NC = 2
_NS = 16
_NW = _NC * _NS
_BPW = _N // _NW
_KCH = 128
_NCH = _BPW // _KCH


def _argmin_body(w_ref, x_ref, idx_ref, minval, minidx):
    j = pl.program_id(1)

    @pl.when(j == 0)
    def _init():
        minval[...] = jnp.full((1, _R), jnp.inf, jnp.float32)
        minidx[...] = jnp.zeros((1, _R), jnp.int32)

    w = w_ref[...]
    xt = x_ref[...].T
    sim = jnp.dot(w * jnp.float32(-2.0), xt,
                  preferred_element_type=jnp.float32)
    s2 = jnp.sum(w * w, axis=1, keepdims=True)
    dist = s2 + sim
    bmin = jnp.min(dist, axis=0, keepdims=True)
    ids = lax.broadcasted_iota(jnp.int32, (_C, _R), 0)
    bidx = jnp.min(jnp.where(dist == bmin, ids, jnp.int32(2 ** 30)),
                   axis=0, keepdims=True) + j * _C
    prev = minval[...]
    better = bmin < prev
    minidx[...] = jnp.where(better, bidx, minidx[...])
    minval[...] = jnp.where(better, bmin, prev)

    @pl.when(j == _NJ - 1)
    def _fin():
        idx_ref[...] = minidx[...].reshape(1, 1, _R)


_argmin_call = pl.pallas_call(
    _argmin_body,
    grid=(_NR, _NJ),
    in_specs=[
        pl.BlockSpec((_C, _D), lambda r, j: (j, 0)),
        pl.BlockSpec((_R, _D), lambda r, j: (r, 0)),
    ],
    out_specs=pl.BlockSpec((1, 1, _R), lambda r, j: (r, 0, 0)),
    out_shape=jax.ShapeDtypeStruct((_NR, 1, _R), jnp.int32),
    scratch_shapes=[
        pltpu.VMEM((1, _R), jnp.float32),
        pltpu.VMEM((1, _R), jnp.int32),
    ],
)


def _gather_body(w_hbm, idx_hbm, x_hbm, q_out, o_out, part_out,
                 idx_v, q_v, x_v, o_v, p_v, sem):
    wid = lax.axis_index("s") * _NC + lax.axis_index("c")
    base = wid * _BPW

    pltpu.sync_copy(idx_hbm.at[pl.ds(wid * _NCH, _NCH), :], idx_v)
    copies = [
        pltpu.async_copy(w_hbm.at[idx_v.at[k]],
                         q_v.at[pl.ds(k * _KCH, _KCH), :], sem)
        for k in range(_NCH)
    ]
    pltpu.sync_copy(x_hbm.at[pl.ds(base, _BPW), :], x_v)
    for cp in copies:
        cp.wait()

    def body(r, acc):
        for c in range(_D // 16):
            sl = pl.ds(c * 16, 16)
            q16 = q_v[r, sl]
            x16 = x_v[r, sl]
            e = q16 - x16
            o_v[r, sl] = x16 + e
            acc = acc + e * e
        return acc

    acc = lax.fori_loop(0, _BPW, body, jnp.zeros((16,), jnp.float32))
    p_v[...] = acc
    pltpu.sync_copy(q_v, q_out.at[pl.ds(base, _BPW), :])
    pltpu.sync_copy(o_v, o_out.at[pl.ds(base, _BPW), :])
    pltpu.sync_copy(p_v, part_out.at[wid])


@functools.cache
def _gather_call():
    return functools.partial(
        pl.kernel,
        out_type=(
            jax.ShapeDtypeStruct((_N, _D), jnp.float32),
            jax.ShapeDtypeStruct((_N, _D), jnp.float32),
            jax.ShapeDtypeStruct((_NW, 16), jnp.float32),
        ),
        mesh=plsc.VectorSubcoreMesh(core_axis_name="c", subcore_axis_name="s",
                                    num_cores=_NC, num_subcores=_NS),
        scratch_types=[
            pltpu.VMEM((_NCH, _KCH), jnp.int32),
            pltpu.VMEM((_BPW, _D), jnp.float32),
            pltpu.VMEM((_BPW, _D), jnp.float32),
            pltpu.VMEM((_BPW, _D), jnp.float32),
            pltpu.VMEM((16,), jnp.float32),
            pltpu.SemaphoreType.DMA,
        ],
        compiler_params=pltpu.CompilerParams(use_tc_tiling_on_sc=False),
    )(_gather_body)


def kernel(x, W):
    x_flat = x.reshape(_N, _D)
    idx = _argmin_call(W, x_flat).reshape(_N)
    idx2 = idx.reshape(_NW * _NCH, _KCH)
    q_flat, o_flat, parts = _gather_call()(W, idx2, x_flat)
    loss = (1.0 + _BETA) * jnp.sum(parts) / (_N * _D)
    quantized = q_flat.reshape(x.shape)
    out = o_flat.reshape(x.shape)
    return (out, quantized, loss)


# speedup vs baseline: 4.2073x; 1.0495x over previous
import functools

import jax
import jax.numpy as jnp
from jax import lax
from jax.experimental import pallas as pl
from jax.experimental.pallas import tpu as pltpu
from jax.experimental.pallas import tpu_sc as plsc

_NE = 8192
_D = 32
_N = 8192
_BETA = 0.25

_R = 2048
_C = 1024
_NR = _N // _R
_NJ = _NE // _C

_NC = 2
_NS = 16
_NW = _NC * _NS
_BPW = _N // _NW
_KCH = 128
_NCH = _BPW // _KCH


def _argmin_body(w_ref, x_ref, idx_ref, loss_ref, minval, minidx, acc):
    r = pl.program_id(0)
    j = pl.program_id(1)

    @pl.when((r == 0) & (j == 0))
    def _init_acc():
        acc[0] = jnp.float32(0.0)

    @pl.when(j == 0)
    def _init():
        minval[...] = jnp.full((1, _R), jnp.inf, jnp.float32)
        minidx[...] = jnp.zeros((1, _R), jnp.int32)

    w = w_ref[...]
    xt = x_ref[...].T
    sim = jnp.dot(w * jnp.float32(-2.0), xt,
                  preferred_element_type=jnp.float32)
    s2 = jnp.sum(w * w, axis=1, keepdims=True)
    dist = s2 + sim
    bmin = jnp.min(dist, axis=0, keepdims=True)
    ids = lax.broadcasted_iota(jnp.int32, (_C, _R), 0)
    bidx = jnp.min(jnp.where(dist == bmin, ids, jnp.int32(2 ** 30)),
                   axis=0, keepdims=True) + j * _C
    prev = minval[...]
    better = bmin < prev
    minidx[...] = jnp.where(better, bidx, minidx[...])
    minval[...] = jnp.where(better, bmin, prev)

    @pl.when(j == _NJ - 1)
    def _fin():
        idx_ref[...] = minidx[...].reshape(1, 1, _R)
        s1 = jnp.sum(xt * xt, axis=0, keepdims=True)
        acc[0] += jnp.sum(minval[...] + s1)

        @pl.when(r == _NR - 1)
        def _loss():
            loss_ref[0] = acc[0] * jnp.float32((1.0 + _BETA) / (_N * _D))


_argmin_call = pl.pallas_call(
    _argmin_body,
    grid=(_NR, _NJ),
    in_specs=[
        pl.BlockSpec((_C, _D), lambda r, j: (j, 0)),
        pl.BlockSpec((_R, _D), lambda r, j: (r, 0)),
    ],
    out_specs=[
        pl.BlockSpec((1, 1, _R), lambda r, j: (r, 0, 0)),
        pl.BlockSpec(memory_space=pltpu.SMEM),
    ],
    out_shape=[
        jax.ShapeDtypeStruct((_NR, 1, _R), jnp.int32),
        jax.ShapeDtypeStruct((1,), jnp.float32),
    ],
    scratch_shapes=[
        pltpu.VMEM((1, _R), jnp.float32),
        pltpu.VMEM((1, _R), jnp.int32),
        pltpu.SMEM((1,), jnp.float32),
    ],
)


def _gather_body(w_hbm, idx_hbm, q_out, idx_v, q_v, sem):
    wid = lax.axis_index("s") * _NC + lax.axis_index("c")
    base = wid * _BPW

    pltpu.sync_copy(idx_hbm.at[pl.ds(wid * _NCH, _NCH), :], idx_v)
    copies = [
        pltpu.async_copy(w_hbm.at[idx_v.at[k]],
                         q_v.at[pl.ds(k * _KCH, _KCH), :], sem)
        for k in range(_NCH)
    ]
    for cp in copies:
        cp.wait()
    pltpu.sync_copy(q_v, q_out.at[pl.ds(base, _BPW), :])


@functools.cache
def _gather_call():
    return functools.partial(
        pl.kernel,
        out_type=jax.ShapeDtypeStruct((_N, _D), jnp.float32),
        mesh=plsc.VectorSubcoreMesh(core_axis_name="c", subcore_axis_name="s",
                                    num_cores=_NC, num_subcores=_NS),
        scratch_types=[
            pltpu.VMEM((_NCH, _KCH), jnp.int32),
            pltpu.VMEM((_BPW, _D), jnp.float32),
            pltpu.SemaphoreType.DMA,
        ],
        compiler_params=pltpu.CompilerParams(use_tc_tiling_on_sc=False),
    )(_gather_body)


def kernel(x, W):
    x_flat = x.reshape(_N, _D)
    idx3, loss1 = _argmin_call(W, x_flat)
    idx2 = idx3.reshape(_NW * _NCH, _KCH)
    q_flat = _gather_call()(W, idx2)
    quantized = q_flat.reshape(x.shape)
    return (quantized, quantized, loss1[0])


# speedup vs baseline: 5.3211x; 1.2647x over previous
import functools

import jax
import jax.numpy as jnp
from jax import lax
from jax.experimental import pallas as pl
from jax.experimental.pallas import tpu as pltpu
from jax.experimental.pallas import tpu_sc as plsc

_NE = 8192
_D = 32
_N = 8192
_BETA = 0.25

_R = 2048
_C = 2048
_NR = _N // _R
_NJ = _NE // _C

_NC = 2
_NS = 16
_NW = _NC * _NS
_BPW = _N // _NW
_KCH = 128
_NCH = _BPW // _KCH


def _argmin_body(w_ref, x_ref, idx_ref, loss_ref, minval, minidx, acc):
    r = pl.program_id(0)
    j = pl.program_id(1)

    @pl.when((r == 0) & (j == 0))
    def _init_acc():
        acc[0] = jnp.float32(0.0)

    @pl.when(j == 0)
    def _init():
        minval[...] = jnp.full((1, _R), jnp.inf, jnp.float32)
        minidx[...] = jnp.zeros((1, _R), jnp.int32)

    w = w_ref[...]
    xt = x_ref[...].T
    sim = jnp.dot(w * jnp.float32(-2.0), xt,
                  preferred_element_type=jnp.float32)
    s2 = jnp.sum(w * w, axis=1, keepdims=True)
    dist = s2 + sim
    bmin = jnp.min(dist, axis=0, keepdims=True)
    bidx = jnp.argmin(dist, axis=0).astype(jnp.int32).reshape(1, _R) + j * _C
    prev = minval[...]
    better = bmin < prev
    minidx[...] = jnp.where(better, bidx, minidx[...])
    minval[...] = jnp.where(better, bmin, prev)

    @pl.when(j == _NJ - 1)
    def _fin():
        idx_ref[...] = minidx[...].reshape(1, 1, _R)
        s1 = jnp.sum(xt * xt, axis=0, keepdims=True)
        acc[0] += jnp.sum(minval[...] + s1)

        @pl.when(r == _NR - 1)
        def _loss():
            loss_ref[0] = acc[0] * jnp.float32((1.0 + _BETA) / (_N * _D))


_argmin_call = pl.pallas_call(
    _argmin_body,
    grid=(_NR, _NJ),
    in_specs=[
        pl.BlockSpec((_C, _D), lambda r, j: (j, 0)),
        pl.BlockSpec((_R, _D), lambda r, j: (r, 0)),
    ],
    out_specs=[
        pl.BlockSpec((1, 1, _R), lambda r, j: (r, 0, 0)),
        pl.BlockSpec(memory_space=pltpu.SMEM),
    ],
    out_shape=[
        jax.ShapeDtypeStruct((_NR, 1, _R), jnp.int32),
        jax.ShapeDtypeStruct((1,), jnp.float32),
    ],
    scratch_shapes=[
        pltpu.VMEM((1, _R), jnp.float32),
        pltpu.VMEM((1, _R), jnp.int32),
        pltpu.SMEM((1,), jnp.float32),
    ],
)


def _gather_body(w_hbm, idx_hbm, q_out, idx_v, q_v, sem):
    wid = lax.axis_index("s") * _NC + lax.axis_index("c")
    base = wid * _BPW

    pltpu.sync_copy(idx_hbm.at[pl.ds(wid * _NCH, _NCH), :], idx_v)
    copies = [
        pltpu.async_copy(w_hbm.at[idx_v.at[k]],
                         q_v.at[pl.ds(k * _KCH, _KCH), :], sem)
        for k in range(_NCH)
    ]
    for cp in copies:
        cp.wait()
    pltpu.sync_copy(q_v, q_out.at[pl.ds(base, _BPW), :])


@functools.cache
def _gather_call():
    return functools.partial(
        pl.kernel,
        out_type=jax.ShapeDtypeStruct((_N, _D), jnp.float32),
        mesh=plsc.VectorSubcoreMesh(core_axis_name="c", subcore_axis_name="s",
                                    num_cores=_NC, num_subcores=_NS),
        scratch_types=[
            pltpu.VMEM((_NCH, _KCH), jnp.int32),
            pltpu.VMEM((_BPW, _D), jnp.float32),
            pltpu.SemaphoreType.DMA,
        ],
        compiler_params=pltpu.CompilerParams(use_tc_tiling_on_sc=False),
    )(_gather_body)


def kernel(x, W):
    x_flat = x.reshape(_N, _D)
    idx3, loss1 = _argmin_call(W, x_flat)
    idx2 = idx3.reshape(_NW * _NCH, _KCH)
    q_flat = _gather_call()(W, idx2)
    quantized = q_flat.reshape(x.shape)
    return (quantized, quantized, loss1[0])


# speedup vs baseline: 6.0513x; 1.1372x over previous
import functools

import jax
import jax.numpy as jnp
from jax import lax
from jax.experimental import pallas as pl
from jax.experimental.pallas import tpu as pltpu
from jax.experimental.pallas import tpu_sc as plsc

_NE = 8192
_D = 32
_N = 8192
_BETA = 0.25

_R = 2048
_C = 2048
_NR = _N // _R
_NJ = _NE // _C

_NC = 2
_NS = 16
_NW = _NC * _NS
_BPW = _N // _NW
_KCH = 128
_NCH = _BPW // _KCH


def _argmin_body(w_ref, x_ref, idx_ref, loss_ref, minval, minidx, acc):
    r = pl.program_id(0)
    j = pl.program_id(1)

    @pl.when((r == 0) & (j == 0))
    def _init_acc():
        acc[0] = jnp.float32(0.0)

    @pl.when(j == 0)
    def _init():
        minval[...] = jnp.full((1, _R), jnp.inf, jnp.float32)
        minidx[...] = jnp.zeros((1, _R), jnp.int32)

    w = w_ref[...]
    xt = x_ref[...].T
    s2 = jnp.sum(w * w, axis=1, keepdims=True)
    aug_w = jnp.concatenate([w * jnp.float32(-2.0), s2], axis=1)
    aug_xt = jnp.concatenate(
        [xt, jnp.ones((1, _R), jnp.float32)], axis=0)
    dist = jnp.dot(aug_w, aug_xt, preferred_element_type=jnp.float32)
    bmin = jnp.min(dist, axis=0, keepdims=True)
    bidx = jnp.argmin(dist, axis=0).astype(jnp.int32).reshape(1, _R) + j * _C
    prev = minval[...]
    better = bmin < prev
    minidx[...] = jnp.where(better, bidx, minidx[...])
    minval[...] = jnp.where(better, bmin, prev)

    @pl.when(j == _NJ - 1)
    def _fin():
        idx_ref[...] = minidx[...].reshape(1, 1, _R)
        s1 = jnp.sum(xt * xt, axis=0, keepdims=True)
        acc[0] += jnp.sum(minval[...] + s1)

        @pl.when(r == _NR - 1)
        def _loss():
            loss_ref[0] = acc[0] * jnp.float32((1.0 + _BETA) / (_N * _D))


_argmin_call = pl.pallas_call(
    _argmin_body,
    grid=(_NR, _NJ),
    in_specs=[
        pl.BlockSpec((_C, _D), lambda r, j: (j, 0)),
        pl.BlockSpec((_R, _D), lambda r, j: (r, 0)),
    ],
    out_specs=[
        pl.BlockSpec((1, 1, _R), lambda r, j: (r, 0, 0)),
        pl.BlockSpec(memory_space=pltpu.SMEM),
    ],
    out_shape=[
        jax.ShapeDtypeStruct((_NR, 1, _R), jnp.int32),
        jax.ShapeDtypeStruct((1,), jnp.float32),
    ],
    scratch_shapes=[
        pltpu.VMEM((1, _R), jnp.float32),
        pltpu.VMEM((1, _R), jnp.int32),
        pltpu.SMEM((1,), jnp.float32),
    ],
)


def _gather_body(w_hbm, idx_hbm, q_out, idx_v, q_v, sem):
    wid = lax.axis_index("s") * _NC + lax.axis_index("c")
    base = wid * _BPW

    pltpu.sync_copy(idx_hbm.at[pl.ds(wid * _NCH, _NCH), :], idx_v)
    copies = [
        pltpu.async_copy(w_hbm.at[idx_v.at[k]],
                         q_v.at[pl.ds(k * _KCH, _KCH), :], sem)
        for k in range(_NCH)
    ]
    for cp in copies:
        cp.wait()
    pltpu.sync_copy(q_v, q_out.at[pl.ds(base, _BPW), :])


@functools.cache
def _gather_call():
    return functools.partial(
        pl.kernel,
        out_type=jax.ShapeDtypeStruct((_N, _D), jnp.float32),
        mesh=plsc.VectorSubcoreMesh(core_axis_name="c", subcore_axis_name="s",
                                    num_cores=_NC, num_subcores=_NS),
        scratch_types=[
            pltpu.VMEM((_NCH, _KCH), jnp.int32),
            pltpu.VMEM((_BPW, _D), jnp.float32),
            pltpu.SemaphoreType.DMA,
        ],
        compiler_params=pltpu.CompilerParams(use_tc_tiling_on_sc=False),
    )(_gather_body)


def kernel(x, W):
    x_flat = x.reshape(_N, _D)
    idx3, loss1 = _argmin_call(W, x_flat)
    idx2 = idx3.reshape(_NW * _NCH, _KCH)
    q_flat = _gather_call()(W, idx2)
    quantized = q_flat.reshape(x.shape)
    return (quantized, quantized, loss1[0])
